# Initial kernel scaffold; baseline (speedup 1.0000x reference)
import jax
import jax.numpy as jnp
from jax.experimental import pallas as pl


def kernel(fields, cells, W_emb, W_fields, W_cells):
    raise NotImplementedError("write your pallas kernel here")


# speedup vs baseline: 10.5928x; 10.5928x over previous
import functools

import jax
import jax.numpy as jnp
from jax import lax
from jax.experimental import pallas as pl
from jax.experimental.pallas import tpu as pltpu
from jax.experimental.pallas import tpu_sc as plsc

NC = 2
NS = 16
NW = NC * NS
NB = 128
L = 4
EMB = 32
HID = 128
TC_BLK = 2048


def _sc_pool_kernel(n_rows_pad, emb):
    rpw = n_rows_pad // NW
    n_blocks = rpw // NB
    mesh = plsc.VectorSubcoreMesh(
        core_axis_name="c", subcore_axis_name="s", num_cores=NC, num_subcores=NS
    )

    @functools.partial(
        pl.kernel,
        out_type=jax.ShapeDtypeStruct((n_rows_pad, emb), jnp.float32),
        mesh=mesh,
        scratch_types=[
            pltpu.VMEM((L, NB), jnp.int32),
            pltpu.VMEM((NB, emb), jnp.float32),
            pltpu.VMEM((NB, emb), jnp.float32),
            pltpu.VMEM((NB, emb), jnp.float32),
            pltpu.VMEM((NB, emb), jnp.float32),
            pltpu.VMEM((NB, emb), jnp.float32),
            pltpu.SemaphoreType.DMA,
        ],
        compiler_params=pltpu.CompilerParams(use_tc_tiling_on_sc=False),
    )
    def sc_kernel(idx_hbm, table_hbm, pooled_hbm, idx_v, r0, r1, r2, r3, out_v, sem):
        wid = lax.axis_index("s") * NC + lax.axis_index("c")
        base = wid * rpw

        def block(b, carry):
            rb = base + b * NB
            pltpu.sync_copy(idx_hbm.at[:, pl.ds(rb, NB)], idx_v)
            cps = [
                pltpu.async_copy(table_hbm.at[idx_v.at[l]], r, sem)
                for l, r in enumerate((r0, r1, r2, r3))
            ]
            for cp in cps:
                cp.wait()

            def pool(i, c):
                for h in range(emb // 16):
                    s = pl.ds(h * 16, 16)
                    out_v[i, s] = r0[i, s] + r1[i, s] + r2[i, s] + r3[i, s]
                return c

            lax.fori_loop(0, NB, pool, 0)
            pltpu.sync_copy(out_v, pooled_hbm.at[pl.ds(rb, NB)])
            return carry

        lax.fori_loop(0, n_blocks, block, 0)

    return sc_kernel


def _tc_proj_kernel(x_ref, wc_ref, wf_ref, o_ref, *, n_cells_blocks):
    pid = pl.program_id(0)
    w = jnp.where(pid < n_cells_blocks, wc_ref[...], wf_ref[...])
    o_ref[...] = jnp.dot(x_ref[...], w, preferred_element_type=jnp.float32)


def kernel(fields, cells, W_emb, W_fields, W_cells):
    B, K, Lf = fields.shape
    _, N, _, Lc = cells.shape
    assert Lf == L and Lc == L
    emb = W_emb.shape[1]
    hid = W_fields.shape[0]

    r_cells = B * N * K
    r_fields = B * K
    r = r_cells + r_fields
    unit = NW * NB
    r_pad = ((r + unit - 1) // unit) * unit

    idx = jnp.concatenate(
        [cells.reshape(r_cells, Lc), fields.reshape(r_fields, Lf)], axis=0
    ).astype(jnp.int32)
    idx = jnp.pad(idx, ((0, r_pad - r), (0, 0))).T

    pooled = _sc_pool_kernel(r_pad, emb)(idx, W_emb)

    wc_t = (W_cells.T * 0.25).astype(jnp.float32)
    wf_t = (W_fields.T * 0.25).astype(jnp.float32)

    assert r_cells % TC_BLK == 0 and r_fields % TC_BLK == 0
    n_cells_blocks = r_cells // TC_BLK
    n_blocks = r // TC_BLK

    out = pl.pallas_call(
        functools.partial(_tc_proj_kernel, n_cells_blocks=n_cells_blocks),
        grid=(n_blocks,),
        in_specs=[
            pl.BlockSpec((TC_BLK, emb), lambda b: (b, 0)),
            pl.BlockSpec((emb, hid), lambda b: (0, 0)),
            pl.BlockSpec((emb, hid), lambda b: (0, 0)),
        ],
        out_specs=pl.BlockSpec((TC_BLK, hid), lambda b: (b, 0)),
        out_shape=jax.ShapeDtypeStruct((r, hid), jnp.float32),
        compiler_params=pltpu.CompilerParams(
            dimension_semantics=("arbitrary",),
        ),
    )(pooled, wc_t, wf_t)

    db_cells_out = out[:r_cells].reshape(B, N, K, hid)
    db_fields_out = out[r_cells:].reshape(B, K, hid)
    return (db_fields_out, db_cells_out)
